# Initial kernel scaffold; baseline (speedup 1.0000x reference)
import jax
import jax.numpy as jnp
from jax.experimental import pallas as pl


def kernel(x1, x2, edge_index, batch, W_l, b_l, W_r, Wc1, bc1, Wc2, bc2, Wc3, bc3, Wc4, bc4, gamma, beta, fW1, fb1, fW2, fb2, fW3, fb3):
    raise NotImplementedError("write your pallas kernel here")


# speedup vs baseline: 11.5112x; 11.5112x over previous
import functools
import jax
import jax.numpy as jnp
from jax import lax
from jax.experimental import pallas as pl
from jax.experimental.pallas import tpu as pltpu
from jax.experimental.pallas import tpu_sc as plsc

N = 100000
E = 3200000
B = 128
L = 600

NC = 2
NS = 16
NW = NC * NS
EPW = E // NW
CHUNK = 80
STEPS = EPW // CHUNK
NPAD = 100096
ROWS_PER_TILE = NPAD // NS


def _edge_agg_body(x1_hbm, src_hbm, dst_hbm, zrows_hbm,
                   sum_out,
                   sum_sh, src_v, dst_v, rows_v, sem):
  c = lax.axis_index("c")
  s = lax.axis_index("s")
  wid = s * NC + c

  r0 = s * ROWS_PER_TILE
  pltpu.sync_copy(zrows_hbm, sum_sh.at[pl.ds(r0, ROWS_PER_TILE)])
  plsc.subcore_barrier()

  base = wid * EPW

  def step(i, _):
    off = base + i * CHUNK
    pltpu.sync_copy(src_hbm.at[pl.ds(off, CHUNK)], src_v)
    pltpu.sync_copy(dst_hbm.at[pl.ds(off, CHUNK)], dst_v)
    pltpu.async_copy(x1_hbm.at[src_v], rows_v, sem).wait()
    pltpu.sync_copy(rows_v, sum_sh.at[dst_v], add=True)
    return ()

  lax.fori_loop(0, STEPS, step, (), unroll=False)
  plsc.subcore_barrier()

  pltpu.sync_copy(sum_sh.at[pl.ds(r0, ROWS_PER_TILE)],
                  sum_out.at[c, pl.ds(r0, ROWS_PER_TILE)])


def _edge_aggregate(x1p, src, dst):
  zrows = jnp.zeros((ROWS_PER_TILE, 16), jnp.float32)
  mesh = plsc.VectorSubcoreMesh(core_axis_name="c", subcore_axis_name="s")
  fn = pl.kernel(
      _edge_agg_body,
      out_type=jax.ShapeDtypeStruct((NC, NPAD, 16), jnp.float32),
      mesh=mesh,
      scratch_types=[
          pltpu.VMEM_SHARED((NPAD, 16), jnp.float32),
          pltpu.VMEM((CHUNK,), jnp.int32),
          pltpu.VMEM((CHUNK,), jnp.int32),
          pltpu.VMEM((CHUNK, 16), jnp.float32),
          pltpu.SemaphoreType.DMA,
      ],
      compiler_params=pltpu.CompilerParams(use_tc_tiling_on_sc=False),
  )
  return fn(x1p, src, dst, zrows)


def _lrelu(x):
  return jnp.where(x >= 0, x, 0.01 * x)


def _conv3(xp, wt, b):
  lout = xp.shape[0] - 2
  y = jnp.dot(xp[0:lout], wt[0], preferred_element_type=jnp.float32, precision=lax.Precision.HIGHEST)
  y = y + jnp.dot(xp[1:lout + 1], wt[1], preferred_element_type=jnp.float32, precision=lax.Precision.HIGHEST)
  y = y + jnp.dot(xp[2:lout + 2], wt[2], preferred_element_type=jnp.float32, precision=lax.Precision.HIGHEST)
  return y + b


def _zpad(x):
  z = jnp.zeros((1, x.shape[1]), jnp.float32)
  return jnp.concatenate([z, x, z], axis=0)


def _ipad(x):
  m = jnp.full((1, x.shape[1]), -jnp.inf, jnp.float32)
  return jnp.concatenate([m, x, m], axis=0)


def _cnn_body(x2t_ref, w1_ref, b1_ref, w2_ref, b2_ref, w3_ref, b3_ref,
              w4_ref, b4_ref, out_ref):
  x = x2t_ref[0]
  a = _lrelu(_conv3(_zpad(x), w1_ref[...], b1_ref[...]))
  a = a.reshape(200, 3, 100).max(axis=1)
  a = _lrelu(_conv3(_zpad(a), w2_ref[...], b2_ref[...]))
  a = _ipad(a)[0:201].reshape(67, 3, 100).max(axis=1)
  a = _lrelu(_conv3(_zpad(a), w3_ref[...], b3_ref[...]))
  a = _ipad(a).reshape(23, 3, 100).max(axis=1)
  a = _lrelu(_conv3(_zpad(a), w4_ref[...], b4_ref[...]))
  a = _ipad(a)[0:24].reshape(8, 3, 100).max(axis=1)
  out_ref[0] = a


def _cnn_branch(x2, Wc1, bc1, Wc2, bc2, Wc3, bc3, Wc4, bc4):
  x2t = jnp.swapaxes(x2, 1, 2)
  wts = [jnp.transpose(W, (2, 1, 0)) for W in (Wc1, Wc2, Wc3, Wc4)]
  bs = [b.reshape(1, -1) for b in (bc1, bc2, bc3, bc4)]
  wspec = pl.BlockSpec(index_map=lambda g: (0, 0, 0))
  bspec = pl.BlockSpec(index_map=lambda g: (0, 0))
  return pl.pallas_call(
      _cnn_body,
      grid=(B,),
      in_specs=[pl.BlockSpec((1, L, 21), lambda g: (g, 0, 0)),
                wspec, bspec, wspec, bspec, wspec, bspec, wspec, bspec],
      out_specs=pl.BlockSpec((1, 8, 100), lambda g: (g, 0, 0)),
      out_shape=jax.ShapeDtypeStruct((B, 8, 100), jnp.float32),
  )(x2t, wts[0], bs[0], wts[1], bs[1], wts[2], bs[2], wts[3], bs[3])


BN_ = 2000
NB = N // BN_


def _final_body(sum2_ref, x1_ref, batch_ref, cnn_ref,
                wl_ref, bl_ref, wr_ref, gam_ref, bet_ref,
                f1_ref, fb1_ref, f2_ref, fb2_ref, f3_ref, fb3_ref,
                out_ref, hp_acc):
  i = pl.program_id(0)

  @pl.when(i == 0)
  def _():
    hp_acc[...] = jnp.zeros_like(hp_acc)

  tot = sum2_ref[0] + sum2_ref[1]
  summed = tot[:, 0:10]
  cnt = tot[:, 10:11]
  mean = summed / jnp.maximum(cnt, 1.0)
  h = (jnp.dot(mean, wl_ref[...], preferred_element_type=jnp.float32)
       + bl_ref[...]
       + jnp.dot(x1_ref[...], wr_ref[...],
                 preferred_element_type=jnp.float32))
  b = batch_ref[0, 0, :]
  oh = (b[:, None] ==
        lax.broadcasted_iota(jnp.int32, (BN_, B), 1)).astype(jnp.float32)
  dn = (((0,), (0,)), ((), ()))
  hp_acc[...] += lax.dot_general(oh, h, dn,
                                 preferred_element_type=jnp.float32, precision=lax.Precision.HIGHEST)

  @pl.when(i == NB - 1)
  def _():
    parts = [hp_acc[...]] + [cnn_ref[:, l, :] for l in range(8)]
    x = jnp.concatenate(parts, axis=1)
    x = x * (1.0 / jnp.sqrt(1.0 + 1e-5)) * gam_ref[...] + bet_ref[...]
    x = jnp.maximum(jnp.dot(x, f1_ref[...],
                            preferred_element_type=jnp.float32)
                    + fb1_ref[...], 0.0)
    x = jnp.maximum(jnp.dot(x, f2_ref[...],
                            preferred_element_type=jnp.float32)
                    + fb2_ref[...], 0.0)
    out_ref[...] = (jnp.dot(x, f3_ref[...],
                            preferred_element_type=jnp.float32)
                    + fb3_ref[...])


def _finalize(sum2, x1, batch, cnn, W_l, b_l, W_r, gamma, beta,
              fW1, fb1, fW2, fb2, fW3, fb3):
  batch_r = batch.reshape(NB, 1, BN_)
  perm = jnp.arange(800).reshape(100, 8).T.reshape(800) + 100
  perm = jnp.concatenate([jnp.arange(100), perm])
  gam_p = gamma[perm].reshape(1, 900)
  bet_p = beta[perm].reshape(1, 900)
  f1_p = fW1.T[perm]
  cspec = lambda r: pl.BlockSpec(index_map=lambda i, _r=r: (0,) * _r)
  return pl.pallas_call(
      _final_body,
      grid=(NB,),
      in_specs=[pl.BlockSpec((NC, BN_, 16), lambda i: (0, i, 0)),
                pl.BlockSpec((BN_, 10), lambda i: (i, 0)),
                pl.BlockSpec((1, 1, BN_), lambda i: (i, 0, 0)),
                cspec(3), cspec(2), cspec(2), cspec(2), cspec(2), cspec(2),
                cspec(2), cspec(2), cspec(2), cspec(2), cspec(2), cspec(2)],
      out_specs=pl.BlockSpec((B, 2), lambda i: (0, 0)),
      out_shape=jax.ShapeDtypeStruct((B, 2), jnp.float32),
      scratch_shapes=[pltpu.VMEM((B, 100), jnp.float32)],
  )(sum2, x1, batch_r, cnn, W_l.T, b_l.reshape(1, -1), W_r.T,
    gam_p, bet_p, f1_p, fb1.reshape(1, -1), fW2.T, fb2.reshape(1, -1),
    fW3.T, fb3.reshape(1, -1))


def kernel(x1, x2, edge_index, batch, W_l, b_l, W_r, Wc1, bc1, Wc2, bc2,
           Wc3, bc3, Wc4, bc4, gamma, beta, fW1, fb1, fW2, fb2, fW3, fb3):
  x1p = jnp.concatenate(
      [x1, jnp.ones((N, 1), jnp.float32), jnp.zeros((N, 5), jnp.float32)],
      axis=1)
  sum2 = _edge_aggregate(x1p, edge_index[0], edge_index[1])
  cnn = _cnn_branch(x2, Wc1, bc1, Wc2, bc2, Wc3, bc3, Wc4, bc4)
  return _finalize(sum2, x1, batch, cnn, W_l, b_l, W_r, gamma, beta,
                   fW1, fb1, fW2, fb2, fW3, fb3)


# speedup vs baseline: 21.8534x; 1.8984x over previous
import functools
import jax
import jax.numpy as jnp
from jax import lax
from jax.experimental import pallas as pl
from jax.experimental.pallas import tpu as pltpu
from jax.experimental.pallas import tpu_sc as plsc

N = 100000
E = 3200000
B = 128
L = 600

NC = 2
NS = 16
NW = NC * NS
EPW = E // NW
CHUNK = 125
SUP = 4
NSUP = EPW // (CHUNK * SUP)
ECHUNKS = E // CHUNK
CPW = EPW // CHUNK
NPAD = 100096
ROWS_PER_TILE = NPAD // NS


def _edge_agg_body(x1_hbm, eidx_hbm, zrows_hbm,
                   sum_out,
                   sum_sh, idx_v, rows_v, isem, gsem, ssem):
  c = lax.axis_index("c")
  s = lax.axis_index("s")
  wid = s * NC + c

  r0 = s * ROWS_PER_TILE
  pltpu.sync_copy(zrows_hbm, sum_sh.at[pl.ds(r0, ROWS_PER_TILE)])
  plsc.subcore_barrier()

  base = wid * CPW

  def fetch_idx(it, slot):
    pltpu.make_async_copy(
        eidx_hbm.at[:, pl.ds(base + it * SUP, SUP), :],
        idx_v.at[slot], isem).start()

  def scatter_descr(slot, j):
    return pltpu.make_async_copy(
        rows_v.at[slot, j], sum_sh.at[idx_v.at[slot, 1, j]], ssem)

  fetch_idx(0, 0)

  def step(it, _):
    p = it % 2
    pltpu.make_async_copy(eidx_hbm.at[:, pl.ds(0, SUP), :],
                          idx_v.at[p], isem).wait()
    for j in range(SUP):
      pltpu.make_async_copy(x1_hbm.at[idx_v.at[p, 0, j]],
                            rows_v.at[p, j], gsem).start()
    for j in range(SUP):
      pltpu.make_async_copy(x1_hbm.at[idx_v.at[p, 0, j]],
                            rows_v.at[p, j], gsem).wait()

    @pl.when(it > 0)
    def _():
      for j in range(SUP):
        scatter_descr(1 - p, j).wait()

    @pl.when(it < NSUP - 1)
    def _():
      fetch_idx(it + 1, 1 - p)

    for j in range(SUP):
      scatter_descr(p, j).start(add=True)
    return ()

  lax.fori_loop(0, NSUP, step, (), unroll=False)
  pf = (NSUP - 1) % 2
  for j in range(SUP):
    scatter_descr(pf, j).wait()
  plsc.subcore_barrier()

  pltpu.sync_copy(sum_sh.at[pl.ds(r0, ROWS_PER_TILE)],
                  sum_out.at[c, pl.ds(r0, ROWS_PER_TILE)])


def _edge_aggregate(x1p, edge_index):
  eidx = edge_index.reshape(2, ECHUNKS, CHUNK)
  zrows = jnp.zeros((ROWS_PER_TILE, 16), jnp.float32)
  mesh = plsc.VectorSubcoreMesh(core_axis_name="c", subcore_axis_name="s")
  fn = pl.kernel(
      _edge_agg_body,
      out_type=jax.ShapeDtypeStruct((NC, NPAD, 16), jnp.float32),
      mesh=mesh,
      scratch_types=[
          pltpu.VMEM_SHARED((NPAD, 16), jnp.float32),
          pltpu.VMEM((2, 2, SUP, CHUNK), jnp.int32),
          pltpu.VMEM((2, SUP, CHUNK, 16), jnp.float32),
          pltpu.SemaphoreType.DMA,
          pltpu.SemaphoreType.DMA,
          pltpu.SemaphoreType.DMA,
      ],
      compiler_params=pltpu.CompilerParams(use_tc_tiling_on_sc=False),
  )
  return fn(x1p, eidx, zrows)


def _lrelu(x):
  return jnp.where(x >= 0, x, 0.01 * x)


def _conv3(xp, wt, b):
  lout = xp.shape[0] - 2
  y = jnp.dot(xp[0:lout], wt[0], preferred_element_type=jnp.float32, precision=lax.Precision.HIGHEST)
  y = y + jnp.dot(xp[1:lout + 1], wt[1], preferred_element_type=jnp.float32, precision=lax.Precision.HIGHEST)
  y = y + jnp.dot(xp[2:lout + 2], wt[2], preferred_element_type=jnp.float32, precision=lax.Precision.HIGHEST)
  return y + b


def _zpad(x):
  z = jnp.zeros((1, x.shape[1]), jnp.float32)
  return jnp.concatenate([z, x, z], axis=0)


def _ipad(x):
  m = jnp.full((1, x.shape[1]), -jnp.inf, jnp.float32)
  return jnp.concatenate([m, x, m], axis=0)


def _cnn_body(x2t_ref, w1_ref, b1_ref, w2_ref, b2_ref, w3_ref, b3_ref,
              w4_ref, b4_ref, out_ref):
  x = x2t_ref[0]
  a = _lrelu(_conv3(_zpad(x), w1_ref[...], b1_ref[...]))
  a = a.reshape(200, 3, 100).max(axis=1)
  a = _lrelu(_conv3(_zpad(a), w2_ref[...], b2_ref[...]))
  a = _ipad(a)[0:201].reshape(67, 3, 100).max(axis=1)
  a = _lrelu(_conv3(_zpad(a), w3_ref[...], b3_ref[...]))
  a = _ipad(a).reshape(23, 3, 100).max(axis=1)
  a = _lrelu(_conv3(_zpad(a), w4_ref[...], b4_ref[...]))
  a = _ipad(a)[0:24].reshape(8, 3, 100).max(axis=1)
  out_ref[0] = a


def _cnn_branch(x2, Wc1, bc1, Wc2, bc2, Wc3, bc3, Wc4, bc4):
  x2t = jnp.swapaxes(x2, 1, 2)
  wts = [jnp.transpose(W, (2, 1, 0)) for W in (Wc1, Wc2, Wc3, Wc4)]
  bs = [b.reshape(1, -1) for b in (bc1, bc2, bc3, bc4)]
  wspec = pl.BlockSpec(index_map=lambda g: (0, 0, 0))
  bspec = pl.BlockSpec(index_map=lambda g: (0, 0))
  return pl.pallas_call(
      _cnn_body,
      grid=(B,),
      in_specs=[pl.BlockSpec((1, L, 21), lambda g: (g, 0, 0)),
                wspec, bspec, wspec, bspec, wspec, bspec, wspec, bspec],
      out_specs=pl.BlockSpec((1, 8, 100), lambda g: (g, 0, 0)),
      out_shape=jax.ShapeDtypeStruct((B, 8, 100), jnp.float32),
  )(x2t, wts[0], bs[0], wts[1], bs[1], wts[2], bs[2], wts[3], bs[3])


BN_ = 2000
NB = N // BN_


def _final_body(sum2_ref, x1_ref, batch_ref, cnn_ref,
                wl_ref, bl_ref, wr_ref, gam_ref, bet_ref,
                f1_ref, fb1_ref, f2_ref, fb2_ref, f3_ref, fb3_ref,
                out_ref, hp_acc):
  i = pl.program_id(0)

  @pl.when(i == 0)
  def _():
    hp_acc[...] = jnp.zeros_like(hp_acc)

  tot = sum2_ref[0] + sum2_ref[1]
  summed = tot[:, 0:10]
  cnt = tot[:, 10:11]
  mean = summed / jnp.maximum(cnt, 1.0)
  h = (jnp.dot(mean, wl_ref[...], preferred_element_type=jnp.float32)
       + bl_ref[...]
       + jnp.dot(x1_ref[...], wr_ref[...],
                 preferred_element_type=jnp.float32))
  b = batch_ref[0, 0, :]
  oh = (b[:, None] ==
        lax.broadcasted_iota(jnp.int32, (BN_, B), 1)).astype(jnp.float32)
  dn = (((0,), (0,)), ((), ()))
  hp_acc[...] += lax.dot_general(oh, h, dn,
                                 preferred_element_type=jnp.float32, precision=lax.Precision.HIGHEST)

  @pl.when(i == NB - 1)
  def _():
    parts = [hp_acc[...]] + [cnn_ref[:, l, :] for l in range(8)]
    x = jnp.concatenate(parts, axis=1)
    x = x * (1.0 / jnp.sqrt(1.0 + 1e-5)) * gam_ref[...] + bet_ref[...]
    x = jnp.maximum(jnp.dot(x, f1_ref[...],
                            preferred_element_type=jnp.float32)
                    + fb1_ref[...], 0.0)
    x = jnp.maximum(jnp.dot(x, f2_ref[...],
                            preferred_element_type=jnp.float32)
                    + fb2_ref[...], 0.0)
    out_ref[...] = (jnp.dot(x, f3_ref[...],
                            preferred_element_type=jnp.float32)
                    + fb3_ref[...])


def _finalize(sum2, x1, batch, cnn, W_l, b_l, W_r, gamma, beta,
              fW1, fb1, fW2, fb2, fW3, fb3):
  batch_r = batch.reshape(NB, 1, BN_)
  perm = jnp.arange(800).reshape(100, 8).T.reshape(800) + 100
  perm = jnp.concatenate([jnp.arange(100), perm])
  gam_p = gamma[perm].reshape(1, 900)
  bet_p = beta[perm].reshape(1, 900)
  f1_p = fW1.T[perm]
  cspec = lambda r: pl.BlockSpec(index_map=lambda i, _r=r: (0,) * _r)
  return pl.pallas_call(
      _final_body,
      grid=(NB,),
      in_specs=[pl.BlockSpec((NC, BN_, 16), lambda i: (0, i, 0)),
                pl.BlockSpec((BN_, 10), lambda i: (i, 0)),
                pl.BlockSpec((1, 1, BN_), lambda i: (i, 0, 0)),
                cspec(3), cspec(2), cspec(2), cspec(2), cspec(2), cspec(2),
                cspec(2), cspec(2), cspec(2), cspec(2), cspec(2), cspec(2)],
      out_specs=pl.BlockSpec((B, 2), lambda i: (0, 0)),
      out_shape=jax.ShapeDtypeStruct((B, 2), jnp.float32),
      scratch_shapes=[pltpu.VMEM((B, 100), jnp.float32)],
  )(sum2, x1, batch_r, cnn, W_l.T, b_l.reshape(1, -1), W_r.T,
    gam_p, bet_p, f1_p, fb1.reshape(1, -1), fW2.T, fb2.reshape(1, -1),
    fW3.T, fb3.reshape(1, -1))


def kernel(x1, x2, edge_index, batch, W_l, b_l, W_r, Wc1, bc1, Wc2, bc2,
           Wc3, bc3, Wc4, bc4, gamma, beta, fW1, fb1, fW2, fb2, fW3, fb3):
  x1p = jnp.concatenate(
      [x1, jnp.ones((N, 1), jnp.float32), jnp.zeros((N, 5), jnp.float32)],
      axis=1)
  sum2 = _edge_aggregate(x1p, edge_index)
  cnn = _cnn_branch(x2, Wc1, bc1, Wc2, bc2, Wc3, bc3, Wc4, bc4)
  return _finalize(sum2, x1, batch, cnn, W_l, b_l, W_r, gamma, beta,
                   fW1, fb1, fW2, fb2, fW3, fb3)


# speedup vs baseline: 21.9179x; 1.0030x over previous
import functools
import jax
import jax.numpy as jnp
from jax import lax
from jax.experimental import pallas as pl
from jax.experimental.pallas import tpu as pltpu
from jax.experimental.pallas import tpu_sc as plsc

N = 100000
E = 3200000
B = 128
L = 600

NC = 2
NS = 16
NW = NC * NS
EPW = E // NW
CHUNK = 125
SUP = 4
NSUP = EPW // (CHUNK * SUP)
ECHUNKS = E // CHUNK
CPW = EPW // CHUNK
NPAD = 100096
ROWS_PER_TILE = NPAD // NS


def _edge_agg_body(x1_hbm, eidx_hbm, zrows_hbm,
                   sum_out,
                   sum_sh, idx_v, rows_v, isem, gsem, ssem):
  c = lax.axis_index("c")
  s = lax.axis_index("s")
  wid = s * NC + c

  r0 = s * ROWS_PER_TILE
  pltpu.sync_copy(zrows_hbm, sum_sh.at[pl.ds(r0, ROWS_PER_TILE)])
  plsc.subcore_barrier()

  base = wid * CPW

  def fetch_idx(it, slot):
    pltpu.make_async_copy(
        eidx_hbm.at[:, pl.ds(base + it * SUP, SUP), :],
        idx_v.at[slot], isem).start()

  def scatter_descr(slot, j):
    return pltpu.make_async_copy(
        rows_v.at[slot, j], sum_sh.at[idx_v.at[slot, 1, j]], ssem)

  fetch_idx(0, 0)

  def step(it, _):
    p = it % 2
    pltpu.make_async_copy(eidx_hbm.at[:, pl.ds(0, SUP), :],
                          idx_v.at[p], isem).wait()
    for j in range(SUP):
      pltpu.make_async_copy(x1_hbm.at[idx_v.at[p, 0, j]],
                            rows_v.at[p, j], gsem).start()
    for j in range(SUP):
      pltpu.make_async_copy(x1_hbm.at[idx_v.at[p, 0, j]],
                            rows_v.at[p, j], gsem).wait()

    @pl.when(it > 0)
    def _():
      for j in range(SUP):
        scatter_descr(1 - p, j).wait()

    @pl.when(it < NSUP - 1)
    def _():
      fetch_idx(it + 1, 1 - p)

    for j in range(SUP):
      scatter_descr(p, j).start(add=True)
    return ()

  lax.fori_loop(0, NSUP, step, (), unroll=False)
  pf = (NSUP - 1) % 2
  for j in range(SUP):
    scatter_descr(pf, j).wait()
  plsc.subcore_barrier()

  pltpu.sync_copy(sum_sh.at[pl.ds(r0, ROWS_PER_TILE)],
                  sum_out.at[c, pl.ds(r0, ROWS_PER_TILE)])


def _edge_aggregate(x1p, edge_index):
  eidx = edge_index.reshape(2, ECHUNKS, CHUNK)
  zrows = jnp.zeros((ROWS_PER_TILE, 16), jnp.float32)
  mesh = plsc.VectorSubcoreMesh(core_axis_name="c", subcore_axis_name="s")
  fn = pl.kernel(
      _edge_agg_body,
      out_type=jax.ShapeDtypeStruct((NC, NPAD, 16), jnp.float32),
      mesh=mesh,
      scratch_types=[
          pltpu.VMEM_SHARED((NPAD, 16), jnp.float32),
          pltpu.VMEM((2, 2, SUP, CHUNK), jnp.int32),
          pltpu.VMEM((2, SUP, CHUNK, 16), jnp.float32),
          pltpu.SemaphoreType.DMA,
          pltpu.SemaphoreType.DMA,
          pltpu.SemaphoreType.DMA,
      ],
      compiler_params=pltpu.CompilerParams(use_tc_tiling_on_sc=False),
  )
  return fn(x1p, eidx, zrows)


def _lrelu(x):
  return jnp.where(x >= 0, x, 0.01 * x)


def _conv3(xp, wt, b):
  lout = xp.shape[0] - 2
  y = jnp.dot(xp[0:lout], wt[0], preferred_element_type=jnp.float32, precision=lax.Precision.HIGHEST)
  y = y + jnp.dot(xp[1:lout + 1], wt[1], preferred_element_type=jnp.float32, precision=lax.Precision.HIGHEST)
  y = y + jnp.dot(xp[2:lout + 2], wt[2], preferred_element_type=jnp.float32, precision=lax.Precision.HIGHEST)
  return y + b


def _zpad(x):
  z = jnp.zeros((1, x.shape[1]), jnp.float32)
  return jnp.concatenate([z, x, z], axis=0)


def _ipad(x):
  m = jnp.full((1, x.shape[1]), -jnp.inf, jnp.float32)
  return jnp.concatenate([m, x, m], axis=0)


GB_ = 8


def _cnn_body(x2t_ref, w1_ref, b1_ref, w2_ref, b2_ref, w3_ref, b3_ref,
              w4_ref, b4_ref, out_ref):
  for g in range(GB_):
    x = x2t_ref[g]
    a = _lrelu(_conv3(_zpad(x), w1_ref[...], b1_ref[...]))
    a = a.reshape(200, 3, 100).max(axis=1)
    a = _lrelu(_conv3(_zpad(a), w2_ref[...], b2_ref[...]))
    a = _ipad(a)[0:201].reshape(67, 3, 100).max(axis=1)
    a = _lrelu(_conv3(_zpad(a), w3_ref[...], b3_ref[...]))
    a = _ipad(a).reshape(23, 3, 100).max(axis=1)
    a = _lrelu(_conv3(_zpad(a), w4_ref[...], b4_ref[...]))
    a = _ipad(a)[0:24].reshape(8, 3, 100).max(axis=1)
    out_ref[g] = a


def _cnn_branch(x2, Wc1, bc1, Wc2, bc2, Wc3, bc3, Wc4, bc4):
  x2t = jnp.swapaxes(x2, 1, 2)
  wts = [jnp.transpose(W, (2, 1, 0)) for W in (Wc1, Wc2, Wc3, Wc4)]
  bs = [b.reshape(1, -1) for b in (bc1, bc2, bc3, bc4)]
  wspec = pl.BlockSpec(index_map=lambda g: (0, 0, 0))
  bspec = pl.BlockSpec(index_map=lambda g: (0, 0))
  return pl.pallas_call(
      _cnn_body,
      grid=(B // GB_,),
      in_specs=[pl.BlockSpec((GB_, L, 21), lambda g: (g, 0, 0)),
                wspec, bspec, wspec, bspec, wspec, bspec, wspec, bspec],
      out_specs=pl.BlockSpec((GB_, 8, 100), lambda g: (g, 0, 0)),
      out_shape=jax.ShapeDtypeStruct((B, 8, 100), jnp.float32),
  )(x2t, wts[0], bs[0], wts[1], bs[1], wts[2], bs[2], wts[3], bs[3])


BN_ = 2000
NB = N // BN_


def _final_body(sum2_ref, x1_ref, batch_ref, cnn_ref,
                wl_ref, bl_ref, wr_ref, gam_ref, bet_ref,
                f1_ref, fb1_ref, f2_ref, fb2_ref, f3_ref, fb3_ref,
                out_ref, hp_acc):
  i = pl.program_id(0)

  @pl.when(i == 0)
  def _():
    hp_acc[...] = jnp.zeros_like(hp_acc)

  tot = sum2_ref[0] + sum2_ref[1]
  summed = tot[:, 0:10]
  cnt = tot[:, 10:11]
  mean = summed / jnp.maximum(cnt, 1.0)
  h = (jnp.dot(mean, wl_ref[...], preferred_element_type=jnp.float32)
       + bl_ref[...]
       + jnp.dot(x1_ref[...], wr_ref[...],
                 preferred_element_type=jnp.float32))
  b = batch_ref[0, 0, :]
  oh = (b[:, None] ==
        lax.broadcasted_iota(jnp.int32, (BN_, B), 1)).astype(jnp.float32)
  dn = (((0,), (0,)), ((), ()))
  hp_acc[...] += lax.dot_general(oh, h, dn,
                                 preferred_element_type=jnp.float32, precision=lax.Precision.HIGHEST)

  @pl.when(i == NB - 1)
  def _():
    parts = [hp_acc[...]] + [cnn_ref[:, l, :] for l in range(8)]
    x = jnp.concatenate(parts, axis=1)
    x = x * (1.0 / jnp.sqrt(1.0 + 1e-5)) * gam_ref[...] + bet_ref[...]
    x = jnp.maximum(jnp.dot(x, f1_ref[...],
                            preferred_element_type=jnp.float32)
                    + fb1_ref[...], 0.0)
    x = jnp.maximum(jnp.dot(x, f2_ref[...],
                            preferred_element_type=jnp.float32)
                    + fb2_ref[...], 0.0)
    out_ref[...] = (jnp.dot(x, f3_ref[...],
                            preferred_element_type=jnp.float32)
                    + fb3_ref[...])


def _finalize(sum2, x1, batch, cnn, W_l, b_l, W_r, gamma, beta,
              fW1, fb1, fW2, fb2, fW3, fb3):
  batch_r = batch.reshape(NB, 1, BN_)
  perm = jnp.arange(800).reshape(100, 8).T.reshape(800) + 100
  perm = jnp.concatenate([jnp.arange(100), perm])
  gam_p = gamma[perm].reshape(1, 900)
  bet_p = beta[perm].reshape(1, 900)
  f1_p = fW1.T[perm]
  cspec = lambda r: pl.BlockSpec(index_map=lambda i, _r=r: (0,) * _r)
  return pl.pallas_call(
      _final_body,
      grid=(NB,),
      in_specs=[pl.BlockSpec((NC, BN_, 16), lambda i: (0, i, 0)),
                pl.BlockSpec((BN_, 10), lambda i: (i, 0)),
                pl.BlockSpec((1, 1, BN_), lambda i: (i, 0, 0)),
                cspec(3), cspec(2), cspec(2), cspec(2), cspec(2), cspec(2),
                cspec(2), cspec(2), cspec(2), cspec(2), cspec(2), cspec(2)],
      out_specs=pl.BlockSpec((B, 2), lambda i: (0, 0)),
      out_shape=jax.ShapeDtypeStruct((B, 2), jnp.float32),
      scratch_shapes=[pltpu.VMEM((B, 100), jnp.float32)],
  )(sum2, x1, batch_r, cnn, W_l.T, b_l.reshape(1, -1), W_r.T,
    gam_p, bet_p, f1_p, fb1.reshape(1, -1), fW2.T, fb2.reshape(1, -1),
    fW3.T, fb3.reshape(1, -1))


def kernel(x1, x2, edge_index, batch, W_l, b_l, W_r, Wc1, bc1, Wc2, bc2,
           Wc3, bc3, Wc4, bc4, gamma, beta, fW1, fb1, fW2, fb2, fW3, fb3):
  x1p = jnp.concatenate(
      [x1, jnp.ones((N, 1), jnp.float32), jnp.zeros((N, 5), jnp.float32)],
      axis=1)
  sum2 = _edge_aggregate(x1p, edge_index)
  cnn = _cnn_branch(x2, Wc1, bc1, Wc2, bc2, Wc3, bc3, Wc4, bc4)
  return _finalize(sum2, x1, batch, cnn, W_l, b_l, W_r, gamma, beta,
                   fW1, fb1, fW2, fb2, fW3, fb3)


# speedup vs baseline: 36.6525x; 1.6723x over previous
import functools
import jax
import jax.numpy as jnp
from jax import lax
from jax.experimental import pallas as pl
from jax.experimental.pallas import tpu as pltpu
from jax.experimental.pallas import tpu_sc as plsc

N = 100000
E = 3200000
B = 128
L = 600

NC = 2
NS = 16
NW = NC * NS
CHUNK = 128
SUP = 4
ECHUNKS = E // CHUNK
CPW = ECHUNKS // NW
NSUP = CPW // SUP
NPAD = 100096
ROWS_PER_TILE = NPAD // NS


def _edge_agg_body(x1_hbm, eidx_hbm, zrows_hbm,
                   sum_out,
                   sum_sh, idx_v, rows_v, isem, gsem, ssem):
  c = lax.axis_index("c")
  s = lax.axis_index("s")
  wid = s * NC + c

  r0 = s * ROWS_PER_TILE
  pltpu.sync_copy(zrows_hbm, sum_sh.at[pl.ds(r0, ROWS_PER_TILE)])
  plsc.subcore_barrier()

  base = wid * CPW + jnp.minimum(wid, 8)

  def fetch_idx(it, slot):
    pltpu.make_async_copy(
        eidx_hbm.at[:, pl.ds(base + it * SUP, SUP), :],
        idx_v.at[slot], isem).start()

  def scatter_descr(slot, j):
    return pltpu.make_async_copy(
        rows_v.at[slot, j], sum_sh.at[idx_v.at[slot, 1, j]], ssem)

  fetch_idx(0, 0)

  def step(it, _):
    p = it % 2
    pltpu.make_async_copy(eidx_hbm.at[:, pl.ds(0, SUP), :],
                          idx_v.at[p], isem).wait()
    for j in range(SUP):
      pltpu.make_async_copy(x1_hbm.at[idx_v.at[p, 0, j]],
                            rows_v.at[p, j], gsem).start()
    for j in range(SUP):
      pltpu.make_async_copy(x1_hbm.at[idx_v.at[p, 0, j]],
                            rows_v.at[p, j], gsem).wait()

    @pl.when(it > 0)
    def _():
      for j in range(SUP):
        scatter_descr(1 - p, j).wait()

    @pl.when(it < NSUP - 1)
    def _():
      fetch_idx(it + 1, 1 - p)

    for j in range(SUP):
      scatter_descr(p, j).start(add=True)
    return ()

  lax.fori_loop(0, NSUP, step, (), unroll=False)
  pf = (NSUP - 1) % 2
  for j in range(SUP):
    scatter_descr(pf, j).wait()

  def tail(ci):
    pltpu.sync_copy(eidx_hbm.at[:, pl.ds(ci, 1), :], idx_v.at[0, :, 0:1])
    pltpu.async_copy(x1_hbm.at[idx_v.at[0, 0, 0]],
                     rows_v.at[0, 0], gsem).wait()
    pltpu.sync_copy(rows_v.at[0, 0], sum_sh.at[idx_v.at[0, 1, 0]], add=True)

  tail(base + NSUP * SUP)

  @pl.when(wid < 8)
  def _():
    tail(base + NSUP * SUP + 1)

  plsc.subcore_barrier()

  pltpu.sync_copy(sum_sh.at[pl.ds(r0, ROWS_PER_TILE)],
                  sum_out.at[c, pl.ds(r0, ROWS_PER_TILE)])


def _detile_body(e_ref, out_ref):
  out_ref[0] = e_ref[0].reshape(ECHUNKS // 25, CHUNK)
  out_ref[1] = e_ref[1].reshape(ECHUNKS // 25, CHUNK)


def _detile_edges(edge_index):
  return pl.pallas_call(
      _detile_body,
      grid=(25,),
      in_specs=[pl.BlockSpec((2, E // 25), lambda i: (0, i))],
      out_specs=pl.BlockSpec((2, ECHUNKS // 25, CHUNK), lambda i: (0, i, 0)),
      out_shape=jax.ShapeDtypeStruct((2, ECHUNKS, CHUNK), jnp.int32),
  )(edge_index)


def _edge_aggregate(x1p, eidx):
  zrows = jnp.zeros((ROWS_PER_TILE, 16), jnp.float32)
  mesh = plsc.VectorSubcoreMesh(core_axis_name="c", subcore_axis_name="s")
  fn = pl.kernel(
      _edge_agg_body,
      out_type=jax.ShapeDtypeStruct((NC, NPAD, 16), jnp.float32),
      mesh=mesh,
      scratch_types=[
          pltpu.VMEM_SHARED((NPAD, 16), jnp.float32),
          pltpu.VMEM((2, 2, SUP, CHUNK), jnp.int32),
          pltpu.VMEM((2, SUP, CHUNK, 16), jnp.float32),
          pltpu.SemaphoreType.DMA,
          pltpu.SemaphoreType.DMA,
          pltpu.SemaphoreType.DMA,
      ],
      compiler_params=pltpu.CompilerParams(use_tc_tiling_on_sc=False),
  )
  return fn(x1p, eidx, zrows)


def _lrelu(x):
  return jnp.where(x >= 0, x, 0.01 * x)


def _conv3(xp, wt, b):
  lout = xp.shape[0] - 2
  y = jnp.dot(xp[0:lout], wt[0], preferred_element_type=jnp.float32, precision=lax.Precision.HIGHEST)
  y = y + jnp.dot(xp[1:lout + 1], wt[1], preferred_element_type=jnp.float32, precision=lax.Precision.HIGHEST)
  y = y + jnp.dot(xp[2:lout + 2], wt[2], preferred_element_type=jnp.float32, precision=lax.Precision.HIGHEST)
  return y + b


def _zpad(x):
  z = jnp.zeros((1, x.shape[1]), jnp.float32)
  return jnp.concatenate([z, x, z], axis=0)


def _ipad(x):
  m = jnp.full((1, x.shape[1]), -jnp.inf, jnp.float32)
  return jnp.concatenate([m, x, m], axis=0)


GB_ = 8


def _cnn_body(x2t_ref, w1_ref, b1_ref, w2_ref, b2_ref, w3_ref, b3_ref,
              w4_ref, b4_ref, out_ref):
  for g in range(GB_):
    x = x2t_ref[g]
    a = _lrelu(_conv3(_zpad(x), w1_ref[...], b1_ref[...]))
    a = a.reshape(200, 3, 100).max(axis=1)
    a = _lrelu(_conv3(_zpad(a), w2_ref[...], b2_ref[...]))
    a = _ipad(a)[0:201].reshape(67, 3, 100).max(axis=1)
    a = _lrelu(_conv3(_zpad(a), w3_ref[...], b3_ref[...]))
    a = _ipad(a).reshape(23, 3, 100).max(axis=1)
    a = _lrelu(_conv3(_zpad(a), w4_ref[...], b4_ref[...]))
    a = _ipad(a)[0:24].reshape(8, 3, 100).max(axis=1)
    out_ref[g] = a


def _cnn_branch(x2, Wc1, bc1, Wc2, bc2, Wc3, bc3, Wc4, bc4):
  x2t = jnp.swapaxes(x2, 1, 2)
  wts = [jnp.transpose(W, (2, 1, 0)) for W in (Wc1, Wc2, Wc3, Wc4)]
  bs = [b.reshape(1, -1) for b in (bc1, bc2, bc3, bc4)]
  wspec = pl.BlockSpec(index_map=lambda g: (0, 0, 0))
  bspec = pl.BlockSpec(index_map=lambda g: (0, 0))
  return pl.pallas_call(
      _cnn_body,
      grid=(B // GB_,),
      in_specs=[pl.BlockSpec((GB_, L, 21), lambda g: (g, 0, 0)),
                wspec, bspec, wspec, bspec, wspec, bspec, wspec, bspec],
      out_specs=pl.BlockSpec((GB_, 8, 100), lambda g: (g, 0, 0)),
      out_shape=jax.ShapeDtypeStruct((B, 8, 100), jnp.float32),
  )(x2t, wts[0], bs[0], wts[1], bs[1], wts[2], bs[2], wts[3], bs[3])


BN_ = 2000
NB = N // BN_


def _final_body(sum2_ref, x1_ref, batch_ref, cnn_ref,
                wl_ref, bl_ref, wr_ref, gam_ref, bet_ref,
                f1_ref, fb1_ref, f2_ref, fb2_ref, f3_ref, fb3_ref,
                out_ref, hp_acc):
  i = pl.program_id(0)

  @pl.when(i == 0)
  def _():
    hp_acc[...] = jnp.zeros_like(hp_acc)

  tot = sum2_ref[0] + sum2_ref[1]
  summed = tot[:, 0:10]
  cnt = tot[:, 10:11]
  mean = summed / jnp.maximum(cnt, 1.0)
  h = (jnp.dot(mean, wl_ref[...], preferred_element_type=jnp.float32)
       + bl_ref[...]
       + jnp.dot(x1_ref[...], wr_ref[...],
                 preferred_element_type=jnp.float32))
  b = batch_ref[0, 0, :]
  oh = (b[:, None] ==
        lax.broadcasted_iota(jnp.int32, (BN_, B), 1)).astype(jnp.float32)
  dn = (((0,), (0,)), ((), ()))
  hp_acc[...] += lax.dot_general(oh, h, dn,
                                 preferred_element_type=jnp.float32, precision=lax.Precision.HIGHEST)

  @pl.when(i == NB - 1)
  def _():
    parts = [hp_acc[...]] + [cnn_ref[:, l, :] for l in range(8)]
    x = jnp.concatenate(parts, axis=1)
    x = x * (1.0 / jnp.sqrt(1.0 + 1e-5)) * gam_ref[...] + bet_ref[...]
    x = jnp.maximum(jnp.dot(x, f1_ref[...],
                            preferred_element_type=jnp.float32)
                    + fb1_ref[...], 0.0)
    x = jnp.maximum(jnp.dot(x, f2_ref[...],
                            preferred_element_type=jnp.float32)
                    + fb2_ref[...], 0.0)
    out_ref[...] = (jnp.dot(x, f3_ref[...],
                            preferred_element_type=jnp.float32)
                    + fb3_ref[...])


def _finalize(sum2, x1, batch, cnn, W_l, b_l, W_r, gamma, beta,
              fW1, fb1, fW2, fb2, fW3, fb3):
  batch_r = batch.reshape(NB, 1, BN_)
  perm = jnp.arange(800).reshape(100, 8).T.reshape(800) + 100
  perm = jnp.concatenate([jnp.arange(100), perm])
  gam_p = gamma[perm].reshape(1, 900)
  bet_p = beta[perm].reshape(1, 900)
  f1_p = fW1.T[perm]
  cspec = lambda r: pl.BlockSpec(index_map=lambda i, _r=r: (0,) * _r)
  return pl.pallas_call(
      _final_body,
      grid=(NB,),
      in_specs=[pl.BlockSpec((NC, BN_, 16), lambda i: (0, i, 0)),
                pl.BlockSpec((BN_, 10), lambda i: (i, 0)),
                pl.BlockSpec((1, 1, BN_), lambda i: (i, 0, 0)),
                cspec(3), cspec(2), cspec(2), cspec(2), cspec(2), cspec(2),
                cspec(2), cspec(2), cspec(2), cspec(2), cspec(2), cspec(2)],
      out_specs=pl.BlockSpec((B, 2), lambda i: (0, 0)),
      out_shape=jax.ShapeDtypeStruct((B, 2), jnp.float32),
      scratch_shapes=[pltpu.VMEM((B, 100), jnp.float32)],
  )(sum2, x1, batch_r, cnn, W_l.T, b_l.reshape(1, -1), W_r.T,
    gam_p, bet_p, f1_p, fb1.reshape(1, -1), fW2.T, fb2.reshape(1, -1),
    fW3.T, fb3.reshape(1, -1))


def kernel(x1, x2, edge_index, batch, W_l, b_l, W_r, Wc1, bc1, Wc2, bc2,
           Wc3, bc3, Wc4, bc4, gamma, beta, fW1, fb1, fW2, fb2, fW3, fb3):
  x1p = jnp.concatenate(
      [x1, jnp.ones((N, 1), jnp.float32), jnp.zeros((N, 5), jnp.float32)],
      axis=1)
  sum2 = _edge_aggregate(x1p, _detile_edges(edge_index))
  cnn = _cnn_branch(x2, Wc1, bc1, Wc2, bc2, Wc3, bc3, Wc4, bc4)
  return _finalize(sum2, x1, batch, cnn, W_l, b_l, W_r, gamma, beta,
                   fW1, fb1, fW2, fb2, fW3, fb3)
